# Initial kernel scaffold; baseline (speedup 1.0000x reference)
import jax
import jax.numpy as jnp
from jax.experimental import pallas as pl


def kernel(discrete, emb_x, emb_y, emb_z):
    raise NotImplementedError("write your pallas kernel here")


# speedup vs baseline: 6.6572x; 6.6572x over previous
import functools

import jax
import jax.numpy as jnp
from jax import lax
from jax.experimental import pallas as pl
from jax.experimental.pallas import tpu as pltpu
from jax.experimental.pallas import tpu_sc as plsc

BINS = 1024
D_Q = 64
DIR_LEN = 21
B, L = 4096, 200
N_TOT = B * L
NW = 32
N_PER = N_TOT // NW
CHUNK = 400
GROUPS = CHUNK // 16
N_CHUNKS = N_PER // CHUNK
TBL = BINS * DIR_LEN
TBL3 = 3 * TBL


def _sc_body(table_hbm, ix_hbm, iy_hbm, iz_hbm, out_hbm,
             table_v, ix0, iy0, iz0, ix1, iy1, iz1, out0, out1,
             sem_in, sem_out):
    wid = lax.axis_index("s") * 2 + lax.axis_index("c")
    base = wid * N_PER
    pltpu.sync_copy(table_hbm, table_v)
    iota = lax.iota(jnp.int32, 16)
    dst0 = iota * D_Q
    zeros = jnp.zeros((16,), jnp.float32)
    srcs = (ix_hbm, iy_hbm, iz_hbm)
    idx_bufs = ((ix0, iy0, iz0), (ix1, iy1, iz1))
    out_bufs = (out0, out1)

    def start_idx(c, b):
        cbase = base + c * CHUNK
        for j in range(3):
            pltpu.async_copy(srcs[j].at[pl.ds(cbase, CHUNK)], idx_bufs[b][j],
                             sem_in.at[b])

    def wait_idx(b):
        for j in range(3):
            pltpu.make_async_copy(srcs[j].at[pl.ds(0, CHUNK)], idx_bufs[b][j],
                                  sem_in.at[b]).wait()

    def start_out(c, b):
        cbase = base + c * CHUNK
        pltpu.async_copy(out_bufs[b],
                         out_hbm.at[pl.ds(cbase * D_Q, CHUNK * D_Q)],
                         sem_out.at[b])

    def wait_out(b):
        pltpu.make_async_copy(out_bufs[b],
                              out_hbm.at[pl.ds(0, CHUNK * D_Q)],
                              sem_out.at[b]).wait()

    def compute(b):
        ixv, iyv, izv = idx_bufs[b]
        ob = out_bufs[b]

        def group_body(g, gcarry):
            goff = g * 16
            ix = ixv[pl.ds(goff, 16)]
            iy = iyv[pl.ds(goff, 16)]
            iz = izv[pl.ds(goff, 16)]
            gx = ix * DIR_LEN
            gy = iy * DIR_LEN + TBL
            gz = iz * DIR_LEN + 2 * TBL
            dst = dst0 + g * (16 * D_Q)
            for off, gv in ((0, gx), (DIR_LEN, gy), (2 * DIR_LEN, gz)):
                vals = [plsc.load_gather(table_v, [gv + d])
                        for d in range(DIR_LEN)]
                for d in range(DIR_LEN):
                    plsc.store_scatter(ob, [dst + off + d], vals[d])
            plsc.store_scatter(ob, [dst + (D_Q - 1)], zeros)
            return gcarry

        lax.fori_loop(0, GROUPS, group_body, 0)

    start_idx(0, 0)
    start_idx(1, 1)

    def chunk_pair(i, carry):
        c0 = i * 2
        for b in range(2):
            c = c0 + b
            wait_idx(b)

            @pl.when(c >= 2)
            def _():
                wait_out(b)

            compute(b)
            start_out(c, b)

            @pl.when(c + 2 < N_CHUNKS)
            def _():
                start_idx(c + 2, b)
        return carry

    lax.fori_loop(0, N_CHUNKS // 2, chunk_pair, 0)
    wait_out(0)
    wait_out(1)


_sc_call = functools.partial(
    pl.kernel,
    mesh=plsc.VectorSubcoreMesh(core_axis_name="c", subcore_axis_name="s"),
    out_type=jax.ShapeDtypeStruct((N_TOT * D_Q,), jnp.float32),
    compiler_params=pltpu.CompilerParams(needs_layout_passes=False),
    scratch_types=[
        pltpu.VMEM((TBL3,), jnp.float32),
        pltpu.VMEM((CHUNK,), jnp.int32),
        pltpu.VMEM((CHUNK,), jnp.int32),
        pltpu.VMEM((CHUNK,), jnp.int32),
        pltpu.VMEM((CHUNK,), jnp.int32),
        pltpu.VMEM((CHUNK,), jnp.int32),
        pltpu.VMEM((CHUNK,), jnp.int32),
        pltpu.VMEM((CHUNK * D_Q,), jnp.float32),
        pltpu.VMEM((CHUNK * D_Q,), jnp.float32),
        pltpu.SemaphoreType.DMA((2,)),
        pltpu.SemaphoreType.DMA((2,)),
    ],
)(_sc_body)


def kernel(discrete, emb_x, emb_y, emb_z):
    ix = discrete[:, :, 0].reshape(-1).astype(jnp.int32)
    iy = discrete[:, :, 1].reshape(-1).astype(jnp.int32)
    iz = discrete[:, :, 2].reshape(-1).astype(jnp.int32)
    table = jnp.concatenate(
        [emb_x.reshape(-1), emb_y.reshape(-1), emb_z.reshape(-1)])
    out = _sc_call(table, ix, iy, iz)
    return out.reshape(B, L, D_Q)


# speedup vs baseline: 11.6215x; 1.7457x over previous
import functools

import jax
import jax.numpy as jnp
from jax import lax
from jax.experimental import pallas as pl
from jax.experimental.pallas import tpu as pltpu
from jax.experimental.pallas import tpu_sc as plsc

BINS = 1024
D_Q = 64
DIR_LEN = 21
B, L = 4096, 200
N_TOT = B * L
NW = 32
N_PER = N_TOT // NW
CHUNK = 400
GROUPS = CHUNK // 16
N_CHUNKS = N_PER // CHUNK
TBL = BINS * DIR_LEN
TBL3 = 3 * TBL


def _sc_body(table_hbm, ix_hbm, iy_hbm, iz_hbm, out_hbm,
             table_v, ix0, iy0, iz0, ix1, iy1, iz1, out0, out1,
             sem_in, sem_out):
    wid = lax.axis_index("s") * 2 + lax.axis_index("c")
    base = wid * N_PER
    pltpu.sync_copy(table_hbm, table_v)
    iota = lax.iota(jnp.int32, 16)
    dst0 = iota * D_Q
    zeros = jnp.zeros((16,), jnp.float32)
    srcs = (ix_hbm, iy_hbm, iz_hbm)
    idx_bufs = ((ix0, iy0, iz0), (ix1, iy1, iz1))
    out_bufs = (out0, out1)

    def start_idx(c, b):
        cbase = base + c * CHUNK
        for j in range(3):
            pltpu.async_copy(srcs[j].at[pl.ds(cbase, CHUNK)], idx_bufs[b][j],
                             sem_in.at[b])

    def wait_idx(b):
        for j in range(3):
            pltpu.make_async_copy(srcs[j].at[pl.ds(0, CHUNK)], idx_bufs[b][j],
                                  sem_in.at[b]).wait()

    def start_out(c, b):
        cbase = base + c * CHUNK
        pltpu.async_copy(out_bufs[b],
                         out_hbm.at[pl.ds(cbase * D_Q, CHUNK * D_Q)],
                         sem_out.at[b])

    def wait_out(b):
        pltpu.make_async_copy(out_bufs[b],
                              out_hbm.at[pl.ds(0, CHUNK * D_Q)],
                              sem_out.at[b]).wait()

    def compute(b):
        ixv, iyv, izv = idx_bufs[b]
        ob = out_bufs[b]

        def group_body(g, gcarry):
            rot = [iota]
            for _ in range(DIR_LEN - 1):
                r = rot[-1]
                rot.append(jnp.where(r == DIR_LEN - 1, 0, r + 1))

            goff = g * 16
            ix = ixv[pl.ds(goff, 16)]
            iy = iyv[pl.ds(goff, 16)]
            iz = izv[pl.ds(goff, 16)]
            bases = (ix * DIR_LEN, iy * DIR_LEN + TBL, iz * DIR_LEN + 2 * TBL)
            dst = dst0 + g * (16 * D_Q)
            dsts = (dst, dst + DIR_LEN, dst + 2 * DIR_LEN)
            for d0 in range(0, DIR_LEN, 7):
                pairs = [(t, d) for d in range(d0, d0 + 7) for t in range(3)]
                vals = [plsc.load_gather(table_v, [bases[t] + rot[d]])
                        for (t, d) in pairs]
                for v, (t, d) in zip(vals, pairs):
                    plsc.store_scatter(ob, [dsts[t] + rot[d]], v)
            plsc.store_scatter(ob, [dst + (D_Q - 1)], zeros)
            return gcarry

        lax.fori_loop(0, GROUPS, group_body, 0)

    start_idx(0, 0)
    start_idx(1, 1)

    def chunk_pair(i, carry):
        c0 = i * 2
        for b in range(2):
            c = c0 + b
            wait_idx(b)

            @pl.when(c >= 2)
            def _():
                wait_out(b)

            compute(b)
            start_out(c, b)

            @pl.when(c + 2 < N_CHUNKS)
            def _():
                start_idx(c + 2, b)
        return carry

    lax.fori_loop(0, N_CHUNKS // 2, chunk_pair, 0)
    wait_out(0)
    wait_out(1)


_sc_call = functools.partial(
    pl.kernel,
    mesh=plsc.VectorSubcoreMesh(core_axis_name="c", subcore_axis_name="s"),
    out_type=jax.ShapeDtypeStruct((N_TOT * D_Q,), jnp.float32),
    compiler_params=pltpu.CompilerParams(needs_layout_passes=False),
    scratch_types=[
        pltpu.VMEM((TBL3,), jnp.float32),
        pltpu.VMEM((CHUNK,), jnp.int32),
        pltpu.VMEM((CHUNK,), jnp.int32),
        pltpu.VMEM((CHUNK,), jnp.int32),
        pltpu.VMEM((CHUNK,), jnp.int32),
        pltpu.VMEM((CHUNK,), jnp.int32),
        pltpu.VMEM((CHUNK,), jnp.int32),
        pltpu.VMEM((CHUNK * D_Q,), jnp.float32),
        pltpu.VMEM((CHUNK * D_Q,), jnp.float32),
        pltpu.SemaphoreType.DMA((2,)),
        pltpu.SemaphoreType.DMA((2,)),
    ],
)(_sc_body)


def kernel(discrete, emb_x, emb_y, emb_z):
    ix = discrete[:, :, 0].reshape(-1).astype(jnp.int32)
    iy = discrete[:, :, 1].reshape(-1).astype(jnp.int32)
    iz = discrete[:, :, 2].reshape(-1).astype(jnp.int32)
    table = jnp.concatenate(
        [emb_x.reshape(-1), emb_y.reshape(-1), emb_z.reshape(-1)])
    out = _sc_call(table, ix, iy, iz)
    return out.reshape(B, L, D_Q)


# speedup vs baseline: 12.3633x; 1.0638x over previous
import functools

import jax
import jax.numpy as jnp
from jax import lax
from jax.experimental import pallas as pl
from jax.experimental.pallas import tpu as pltpu
from jax.experimental.pallas import tpu_sc as plsc

BINS = 1024
D_Q = 64
DIR_LEN = 21
B, L = 4096, 200
N_TOT = B * L
NW = 32
N_PER = N_TOT // NW
CHUNK = 400
GROUPS = CHUNK // 16
N_CHUNKS = N_PER // CHUNK
TBL = BINS * DIR_LEN
TBL3 = 3 * TBL


def _sc_body(table_hbm, rot_hbm, ix_hbm, iy_hbm, iz_hbm, out_hbm,
             table_v, rot_v, ix0, iy0, iz0, ix1, iy1, iz1, out0, out1,
             sem_in, sem_out):
    wid = lax.axis_index("s") * 2 + lax.axis_index("c")
    base = wid * N_PER
    pltpu.sync_copy(table_hbm, table_v)
    pltpu.sync_copy(rot_hbm, rot_v)
    iota = lax.iota(jnp.int32, 16)
    dst0 = iota * D_Q
    zeros = jnp.zeros((16,), jnp.float32)
    srcs = (ix_hbm, iy_hbm, iz_hbm)
    idx_bufs = ((ix0, iy0, iz0), (ix1, iy1, iz1))
    out_bufs = (out0, out1)

    def start_idx(c, b):
        cbase = base + c * CHUNK
        for j in range(3):
            pltpu.async_copy(srcs[j].at[pl.ds(cbase, CHUNK)], idx_bufs[b][j],
                             sem_in.at[b])

    def wait_idx(b):
        for j in range(3):
            pltpu.make_async_copy(srcs[j].at[pl.ds(0, CHUNK)], idx_bufs[b][j],
                                  sem_in.at[b]).wait()

    def start_out(c, b):
        cbase = base + c * CHUNK
        pltpu.async_copy(out_bufs[b],
                         out_hbm.at[pl.ds(cbase * D_Q, CHUNK * D_Q)],
                         sem_out.at[b])

    def wait_out(b):
        pltpu.make_async_copy(out_bufs[b],
                              out_hbm.at[pl.ds(0, CHUNK * D_Q)],
                              sem_out.at[b]).wait()

    def compute(b):
        ixv, iyv, izv = idx_bufs[b]
        ob = out_bufs[b]

        def group_body(g, gcarry):
            rot = [rot_v[pl.ds(16 * d, 16)] for d in range(DIR_LEN)]

            goff = g * 16
            ix = ixv[pl.ds(goff, 16)]
            iy = iyv[pl.ds(goff, 16)]
            iz = izv[pl.ds(goff, 16)]
            bases = (ix * DIR_LEN, iy * DIR_LEN + TBL, iz * DIR_LEN + 2 * TBL)
            dst = dst0 + g * (16 * D_Q)
            dsts = (dst, dst + DIR_LEN, dst + 2 * DIR_LEN)
            for d0 in range(0, DIR_LEN, 7):
                pairs = [(t, d) for d in range(d0, d0 + 7) for t in range(3)]
                vals = [plsc.load_gather(table_v, [bases[t] + rot[d]])
                        for (t, d) in pairs]
                for v, (t, d) in zip(vals, pairs):
                    plsc.store_scatter(ob, [dsts[t] + rot[d]], v)
            plsc.store_scatter(ob, [dst + (D_Q - 1)], zeros)
            return gcarry

        lax.fori_loop(0, GROUPS, group_body, 0)

    start_idx(0, 0)
    start_idx(1, 1)

    def chunk_pair(i, carry):
        c0 = i * 2
        for b in range(2):
            c = c0 + b
            wait_idx(b)

            @pl.when(c >= 2)
            def _():
                wait_out(b)

            compute(b)
            start_out(c, b)

            @pl.when(c + 2 < N_CHUNKS)
            def _():
                start_idx(c + 2, b)
        return carry

    lax.fori_loop(0, N_CHUNKS // 2, chunk_pair, 0)
    wait_out(0)
    wait_out(1)


_sc_call = functools.partial(
    pl.kernel,
    mesh=plsc.VectorSubcoreMesh(core_axis_name="c", subcore_axis_name="s"),
    out_type=jax.ShapeDtypeStruct((N_TOT * D_Q,), jnp.float32),
    compiler_params=pltpu.CompilerParams(needs_layout_passes=False),
    scratch_types=[
        pltpu.VMEM((TBL3,), jnp.float32),
        pltpu.VMEM((16 * DIR_LEN,), jnp.int32),
        pltpu.VMEM((CHUNK,), jnp.int32),
        pltpu.VMEM((CHUNK,), jnp.int32),
        pltpu.VMEM((CHUNK,), jnp.int32),
        pltpu.VMEM((CHUNK,), jnp.int32),
        pltpu.VMEM((CHUNK,), jnp.int32),
        pltpu.VMEM((CHUNK,), jnp.int32),
        pltpu.VMEM((CHUNK * D_Q,), jnp.float32),
        pltpu.VMEM((CHUNK * D_Q,), jnp.float32),
        pltpu.SemaphoreType.DMA((2,)),
        pltpu.SemaphoreType.DMA((2,)),
    ],
)(_sc_body)


def kernel(discrete, emb_x, emb_y, emb_z):
    ix = discrete[:, :, 0].reshape(-1).astype(jnp.int32)
    iy = discrete[:, :, 1].reshape(-1).astype(jnp.int32)
    iz = discrete[:, :, 2].reshape(-1).astype(jnp.int32)
    table = jnp.concatenate(
        [emb_x.reshape(-1), emb_y.reshape(-1), emb_z.reshape(-1)])
    rot = jnp.array([(d + i) % DIR_LEN for d in range(DIR_LEN)
                     for i in range(16)], dtype=jnp.int32)
    out = _sc_call(table, rot, ix, iy, iz)
    return out.reshape(B, L, D_Q)
